# Initial kernel scaffold; baseline (speedup 1.0000x reference)
import functools

import jax
import jax.numpy as jnp
from jax import lax
from jax.experimental import pallas as pl
from jax.experimental.pallas import tpu as pltpu
from jax.experimental.pallas import tpu_sc as plsc

BATCH = 16384
SEQ = 200
DIM = 64
NCLS = 4

NC = 2
NS = 16
NW = NC * NS
BPW = BATCH // NW
G = 2
GI = G * SEQ
NGRP = BPW // G
CHUNKS = [(0, 128), (128, 128), (256, 128), (384, 16)]

_mesh = plsc.VectorSubcoreMesh(core_axis_name="c", subcore_axis_name="s")


@functools.partial(
    pl.kernel,
    mesh=_mesh,
    out_type=jax.ShapeDtypeStruct((BATCH, NCLS), jnp.float32),
    scratch_types=[
        pltpu.VMEM((GI,), jnp.int32),
        pltpu.VMEM((GI, DIM), jnp.float32),
        pltpu.VMEM((BPW, NCLS), jnp.float32),
        pltpu.VMEM((NCLS, DIM), jnp.float32),
        pltpu.VMEM((8,), jnp.float32),
        pltpu.SemaphoreType.DMA,
    ],
)
def _embed_fc_kernel(text_hbm, table_hbm, fcw_hbm, fcb_hbm, out_hbm,
                     idx_v, rows_v, out_v, fcw_v, fcb_v, sem):
    wid = lax.axis_index("s") * NC + lax.axis_index("c")
    base = wid * BPW

    pltpu.sync_copy(fcw_hbm, fcw_v)
    pltpu.sync_copy(fcb_hbm, fcb_v)

    inv = jnp.float32(1.0 / SEQ)

    def group_body(g, carry):
        row0 = base + g * G
        pltpu.sync_copy(text_hbm.at[pl.ds(row0 * SEQ, GI)], idx_v)
        handles = [
            pltpu.async_copy(
                table_hbm.at[idx_v.at[pl.ds(off, ln)]],
                rows_v.at[pl.ds(off, ln)],
                sem,
            )
            for off, ln in CHUNKS
        ]
        for h in handles:
            h.wait()

        def bag_body(b, carry2):
            r0 = b * SEQ

            def acc_body(i, accs):
                a0, a1, a2, a3 = accs
                r = r0 + i
                a0 = a0 + rows_v[r, pl.ds(0, 16)]
                a1 = a1 + rows_v[r, pl.ds(16, 16)]
                a2 = a2 + rows_v[r, pl.ds(32, 16)]
                a3 = a3 + rows_v[r, pl.ds(48, 16)]
                return (a0, a1, a2, a3)

            z = jnp.zeros((16,), jnp.float32)
            a0, a1, a2, a3 = lax.fori_loop(0, SEQ, acc_body, (z, z, z, z))
            m0 = a0 * inv
            m1 = a1 * inv
            m2 = a2 * inv
            m3 = a3 * inv
            for c in range(NCLS):
                w0 = fcw_v[c, pl.ds(0, 16)]
                w1 = fcw_v[c, pl.ds(16, 16)]
                w2 = fcw_v[c, pl.ds(32, 16)]
                w3 = fcw_v[c, pl.ds(48, 16)]
                dot = jnp.sum(m0 * w0 + m1 * w1 + m2 * w2 + m3 * w3)
                out_v[g * G + b, c] = dot + fcb_v[c]
            return carry2

        lax.fori_loop(0, G, bag_body, 0)
        return carry

    lax.fori_loop(0, NGRP, group_body, 0)
    pltpu.sync_copy(out_v, out_hbm.at[pl.ds(base, BPW)])


def kernel(text, emb_table, fc_w, fc_b):
    text_flat = text.reshape(-1).astype(jnp.int32)
    fcb_pad = jnp.pad(fc_b, (0, 8 - NCLS)).astype(jnp.float32)
    return _embed_fc_kernel(text_flat, emb_table, fc_w, fcb_pad)


# speedup vs baseline: 2.1407x; 2.1407x over previous
import functools

import jax
import jax.numpy as jnp
from jax import lax
from jax.experimental import pallas as pl
from jax.experimental.pallas import tpu as pltpu
from jax.experimental.pallas import tpu_sc as plsc

BATCH = 16384
SEQ = 200
DIM = 64
NCLS = 4

NC = 2
NS = 16
NW = NC * NS
BPW = BATCH // NW
G = 4
GI = G * SEQ
NGRP = BPW // G
CHUNKS = [(o, min(128, GI - o)) for o in range(0, GI, 128)]

_mesh = plsc.VectorSubcoreMesh(core_axis_name="c", subcore_axis_name="s")


@functools.partial(
    pl.kernel,
    mesh=_mesh,
    compiler_params=pltpu.CompilerParams(
        use_tc_tiling_on_sc=False, needs_layout_passes=False),
    out_type=jax.ShapeDtypeStruct((BATCH * NCLS,), jnp.float32),
    scratch_types=[
        pltpu.VMEM((GI,), jnp.int32),
        pltpu.VMEM((GI, DIM), jnp.float32),
        pltpu.VMEM((BPW * NCLS,), jnp.float32),
        pltpu.VMEM((NCLS * DIM,), jnp.float32),
        pltpu.VMEM((16,), jnp.float32),
        pltpu.VMEM((G * DIM,), jnp.float32),
        pltpu.VMEM((DIM, 16), jnp.float32),
        pltpu.SemaphoreType.DMA,
    ],
)
def _embed_fc_kernel(text_hbm, table_hbm, fcw_hbm, fcb_hbm, out_hbm,
                     idx_v, rows_v, out_v, fcw_v, fcb_v, asum_v, wexp_v, sem):
    wid = lax.axis_index("s") * NC + lax.axis_index("c")
    base = wid * BPW

    pltpu.sync_copy(fcw_hbm, fcw_v)
    pltpu.sync_copy(fcb_hbm, fcb_v)

    inv = jnp.float32(1.0 / SEQ)
    bias = fcb_v[...]
    lane = lax.iota(jnp.int32, 16)
    lane_bag = lane // 4
    lane_cls = lane % 4
    for d in range(DIM):
        wexp_v[d, pl.ds(0, 16)] = plsc.load_gather(fcw_v, [lane_cls * DIM + d])

    def cluster_body(g, carry):
        row0 = base + g * G
        pltpu.sync_copy(text_hbm.at[pl.ds(row0 * SEQ, GI)], idx_v)
        handles = [
            pltpu.async_copy(
                table_hbm.at[idx_v.at[pl.ds(off, ln)]],
                rows_v.at[pl.ds(off, ln)],
                sem,
            )
            for off, ln in CHUNKS
        ]
        for h in handles:
            h.wait()

        for b in range(G):
            r0 = b * SEQ

            def acc_body(i, accs):
                a0, a1, a2, a3 = accs
                r = r0 + i
                a0 = a0 + rows_v[r, pl.ds(0, 16)]
                a1 = a1 + rows_v[r, pl.ds(16, 16)]
                a2 = a2 + rows_v[r, pl.ds(32, 16)]
                a3 = a3 + rows_v[r, pl.ds(48, 16)]
                return (a0, a1, a2, a3)

            z = jnp.zeros((16,), jnp.float32)
            a0, a1, a2, a3 = lax.fori_loop(0, SEQ, acc_body, (z, z, z, z))
            asum_v[pl.ds(b * DIM, 16)] = a0
            asum_v[pl.ds(b * DIM + 16, 16)] = a1
            asum_v[pl.ds(b * DIM + 32, 16)] = a2
            asum_v[pl.ds(b * DIM + 48, 16)] = a3

        def fc_body(d, acc):
            av = plsc.load_gather(asum_v, [lane_bag * DIM + d])
            return acc + av * wexp_v[d, pl.ds(0, 16)]

        res = lax.fori_loop(0, DIM, fc_body, jnp.zeros((16,), jnp.float32))
        out_v[pl.ds(g * 16, 16)] = res * inv + bias
        return carry

    lax.fori_loop(0, NGRP, cluster_body, 0)
    pltpu.sync_copy(out_v, out_hbm.at[pl.ds(base * NCLS, BPW * NCLS)])


def kernel(text, emb_table, fc_w, fc_b):
    text_flat = text.reshape(-1).astype(jnp.int32)
    fcw_flat = fc_w.reshape(-1).astype(jnp.float32)
    fcb_tiled = jnp.tile(fc_b.astype(jnp.float32), 4)
    out = _embed_fc_kernel(text_flat, emb_table, fcw_flat, fcb_tiled)
    return out.reshape(BATCH, NCLS)
